# baseline (speedup 1.0000x reference)
import functools

import jax
import jax.numpy as jnp
from jax import lax
from jax.experimental import pallas as pl
from jax.experimental.pallas import tpu as pltpu
from jax.experimental.pallas import tpu_sc as plsc

N = 10000
E = 320000
EC = 128
NC = 2
NT = 16
R = 2560
EPAD = R * EC
NPAD = NT * 640
SL = NPAD // NT

_mesh = plsc.VectorSubcoreMesh(core_axis_name="c", subcore_axis_name="s")
_sc_params = pltpu.CompilerParams(use_tc_tiling_on_sc=False)


def _deg_body(dst2d_hbm, degp_hbm, acc, zbuf, didx, ones):
  c = lax.axis_index("c")
  s = lax.axis_index("s")
  w = s * NC + c
  nrows = R // (NC * NT)

  def zb(i, _):
    zbuf[pl.ds(i * 16, 16)] = jnp.zeros((16,), jnp.float32)
    return 0
  lax.fori_loop(0, SL // 16, zb, 0)

  def ob(i, _):
    ones[pl.ds(i * 16, 16)] = jnp.ones((16,), jnp.float32)
    return 0
  lax.fori_loop(0, EC // 16, ob, 0)

  pltpu.sync_copy(dst2d_hbm.at[pl.ds(w * nrows, nrows), :], didx)
  pltpu.sync_copy(zbuf, acc.at[pl.ds(s * SL, SL)])
  plsc.subcore_barrier()

  def body(j, _):
    pltpu.sync_copy(ones, acc.at[didx.at[j]], add=True)
    return 0
  lax.fori_loop(0, nrows, body, 0)

  plsc.subcore_barrier()
  pltpu.sync_copy(acc.at[pl.ds(s * SL, SL)],
                  degp_hbm.at[c, 0, pl.ds(s * SL, SL)])


_deg_kernel = pl.kernel(
    _deg_body,
    out_type=jax.ShapeDtypeStruct((NC, 1, NPAD), jnp.float32),
    mesh=_mesh,
    compiler_params=_sc_params,
    scratch_types=[
        pltpu.VMEM_SHARED((NPAD,), jnp.float32),
        pltpu.VMEM((SL,), jnp.float32),
        pltpu.VMEM((R // (NC * NT), EC), jnp.int32),
        pltpu.VMEM((EC,), jnp.float32),
    ],
)


def _scatter_body(NB, y2_hbm, idx3_hbm, dst2d_hbm, out_hbm,
                  acc, sidx, didx, rows, gsem, ssem):
  c = lax.axis_index("c")
  s = lax.axis_index("s")
  nrows = R // NT
  base = s * nrows

  pltpu.sync_copy(idx3_hbm.at[c, pl.ds(base, nrows), :], sidx)
  pltpu.sync_copy(dst2d_hbm.at[pl.ds(base, nrows), :], didx)
  pltpu.sync_copy(y2_hbm.at[pl.ds(c * NPAD + s * SL, SL), :],
                  acc.at[pl.ds(s * SL, SL), :])
  plsc.subcore_barrier()

  def gstart(j, q):
    pltpu.async_copy(y2_hbm.at[sidx.at[j]], rows.at[q], gsem)

  def gwait(j, q):
    pltpu.make_async_copy(y2_hbm.at[sidx.at[j]], rows.at[q], gsem).wait()

  def sstart(j, q):
    pltpu.async_copy(rows.at[q], acc.at[didx.at[j]], ssem, add=True)

  def swait(j, q):
    pltpu.make_async_copy(rows.at[q], acc.at[didx.at[j]], ssem).wait()

  for q in range(NB):
    gstart(q, q)

  def body(i, _):
    jA = 2 * NB * i
    jB = jA + NB
    for q in range(NB):
      gwait(jA + q, q)
      sstart(jA + q, q)
    for q in range(NB):
      @pl.when(i > 0)
      def _():
        swait(jA - NB + q, NB + q)

      gstart(jB + q, NB + q)
    for q in range(NB):
      gwait(jB + q, NB + q)
      sstart(jB + q, NB + q)
    for q in range(NB):
      swait(jA + q, q)

      @pl.when(jB + NB + q < nrows)
      def _():
        gstart(jB + NB + q, q)
    return 0
  lax.fori_loop(0, nrows // (2 * NB), body, 0)
  for q in range(NB):
    swait(nrows - NB + q, NB + q)

  plsc.subcore_barrier()
  pltpu.sync_copy(acc.at[pl.ds(s * SL, SL), :],
                  out_hbm.at[c, pl.ds(s * SL, SL), :])


@functools.cache
def _scatter_kernel(dh):
  nb = 4 if dh <= 32 else 2
  return pl.kernel(
      functools.partial(_scatter_body, nb),
      out_type=jax.ShapeDtypeStruct((NC, NPAD, dh), jnp.float32),
      mesh=_mesh,
      compiler_params=_sc_params,
      scratch_types=[
          pltpu.VMEM_SHARED((NPAD, dh), jnp.float32),
          pltpu.VMEM((R // NT, EC), jnp.int32),
          pltpu.VMEM((R // NT, EC), jnp.int32),
          pltpu.VMEM((2 * nb, EC, dh), jnp.float32),
          pltpu.SemaphoreType.DMA,
          pltpu.SemaphoreType.DMA,
      ],
  )


BR = 1000


def _mm1_body(x_ref, d0_ref, d1_ref, w_ref, o_ref):
  dinv = lax.rsqrt(d0_ref[...] + d1_ref[...] + 1.0)
  y = jnp.dot(x_ref[...], w_ref[...],
              preferred_element_type=jnp.float32) * dinv
  h = y.shape[1] // 2
  o_ref[0] = y[:, :h]
  o_ref[1] = y[:, h:]


def _mid_body(aL_ref, aR_ref, d0_ref, d1_ref, bL_ref, bR_ref, wt_ref, wb_ref,
              o_ref):
  dinv = lax.rsqrt(d0_ref[...] + d1_ref[...] + 1.0)
  hL = jnp.maximum(aL_ref[0] * dinv + bL_ref[...], 0.0)
  hR = jnp.maximum(aR_ref[0] * dinv + bR_ref[...], 0.0)
  y = (jnp.dot(hL, wt_ref[...], preferred_element_type=jnp.float32)
       + jnp.dot(hR, wb_ref[...], preferred_element_type=jnp.float32)) * dinv
  h = y.shape[1] // 2
  o_ref[0] = y[:, :h]
  o_ref[1] = y[:, h:]


def _fin_body(aL_ref, aR_ref, d0_ref, d1_ref, bL_ref, bR_ref, o_ref):
  dinv = lax.rsqrt(d0_ref[...] + d1_ref[...] + 1.0)
  o_ref[...] = jnp.concatenate(
      [aL_ref[0] * dinv + bL_ref[...], aR_ref[0] * dinv + bR_ref[...]],
      axis=1)


def _row_spec(width):
  return pl.BlockSpec((BR, width), lambda i: (i, 0))


def _half_spec(half, width):
  return pl.BlockSpec((1, BR, width), lambda i, _h=half: (_h, i, 0))


def _out3_spec(width):
  return pl.BlockSpec((2, BR, width), lambda i: (0, i, 0))


def _whole_spec(shape):
  return pl.BlockSpec(shape, lambda i: tuple(0 for _ in shape))


def _mm1(x, d0, d1, w):
  fout = w.shape[1]
  return pl.pallas_call(
      _mm1_body,
      grid=(N // BR,),
      in_specs=[_row_spec(x.shape[1]), _row_spec(1), _row_spec(1),
                _whole_spec(w.shape)],
      out_specs=_out3_spec(fout // 2),
      out_shape=jax.ShapeDtypeStruct((2, NPAD, fout // 2), jnp.float32),
  )(x, d0, d1, w)


def _mid(a, d0, d1, bL, bR, wt, wb):
  dh = a.shape[2]
  fout = wt.shape[1]
  return pl.pallas_call(
      _mid_body,
      grid=(N // BR,),
      in_specs=[_half_spec(0, dh), _half_spec(1, dh),
                _row_spec(1), _row_spec(1),
                _whole_spec(bL.shape), _whole_spec(bR.shape),
                _whole_spec(wt.shape), _whole_spec(wb.shape)],
      out_specs=_out3_spec(fout // 2),
      out_shape=jax.ShapeDtypeStruct((2, NPAD, fout // 2), jnp.float32),
  )(a, a, d0, d1, bL, bR, wt, wb)


def _fin(a, d0, d1, bL, bR):
  dh = a.shape[2]
  return pl.pallas_call(
      _fin_body,
      grid=(N // BR,),
      in_specs=[_half_spec(0, dh), _half_spec(1, dh),
                _row_spec(1), _row_spec(1),
                _whole_spec(bL.shape), _whole_spec(bR.shape)],
      out_specs=_row_spec(2 * dh),
      out_shape=jax.ShapeDtypeStruct((N, 2 * dh), jnp.float32),
  )(a, a, d0, d1, bL, bR)


@jax.jit
def _run(x, edge_index, W1, b1, W2, b2, W3, b3, W4, b4):
  src = edge_index[0]
  dst = edge_index[1]
  src_p = jnp.concatenate([src, jnp.zeros((EPAD - E,), jnp.int32)])
  dst_p = jnp.concatenate([dst, jnp.full((EPAD - E,), N, jnp.int32)])
  src2d = src_p.reshape(R, EC)
  dst2d = dst_p.reshape(R, EC)
  idx3 = jnp.stack([src2d, src2d + NPAD])

  degp = _deg_kernel(dst2d)
  d0 = degp[0, 0, :N, None]
  d1 = degp[1, 0, :N, None]

  def scatter(y3):
    dh = y3.shape[2]
    return _scatter_kernel(dh)(y3.reshape(2 * NPAD, dh), idx3, dst2d)

  a = scatter(_mm1(x, d0, d1, W1))
  a = scatter(_mid(a, d0, d1, b1[None, :32], b1[None, 32:], W2[:32], W2[32:]))
  a = scatter(_mid(a, d0, d1, b2[None, :16], b2[None, 16:], W3[:16], W3[16:]))
  a = scatter(_mid(a, d0, d1, b3[None, :32], b3[None, 32:], W4[:32], W4[32:]))
  return _fin(a, d0, d1, b4[None, :64], b4[None, 64:])


def kernel(x, edge_index, W1, b1, W2, b2, W3, b3, W4, b4):
  return _run(x, edge_index, W1, b1, W2, b2, W3, b3, W4, b4)


# speedup vs baseline: 1.8092x; 1.8092x over previous
import functools

import jax
import jax.numpy as jnp
from jax import lax
from jax.experimental import pallas as pl
from jax.experimental.pallas import tpu as pltpu
from jax.experimental.pallas import tpu_sc as plsc

N = 10000
E = 320000
EC = 128
NC = 2
NT = 16
R = 2560
EPAD = R * EC
NPAD = NT * 640
SL = NPAD // NT

_mesh = plsc.VectorSubcoreMesh(core_axis_name="c", subcore_axis_name="s")
_sc_params = pltpu.CompilerParams(use_tc_tiling_on_sc=False)


def _deg_body(dst2d_hbm, degp_hbm, acc, zbuf, didx, ones):
  c = lax.axis_index("c")
  s = lax.axis_index("s")
  w = s * NC + c
  nrows = R // (NC * NT)

  def zb(i, _):
    zbuf[pl.ds(i * 16, 16)] = jnp.zeros((16,), jnp.float32)
    return 0
  lax.fori_loop(0, SL // 16, zb, 0)

  def ob(i, _):
    ones[pl.ds(i * 16, 16)] = jnp.ones((16,), jnp.float32)
    return 0
  lax.fori_loop(0, EC // 16, ob, 0)

  pltpu.sync_copy(dst2d_hbm.at[pl.ds(w * nrows, nrows), :], didx)
  pltpu.sync_copy(zbuf, acc.at[pl.ds(s * SL, SL)])
  plsc.subcore_barrier()

  def body(j, _):
    pltpu.sync_copy(ones, acc.at[didx.at[j]], add=True)
    return 0
  lax.fori_loop(0, nrows, body, 0)

  plsc.subcore_barrier()
  pltpu.sync_copy(acc.at[pl.ds(s * SL, SL)],
                  degp_hbm.at[c, 0, pl.ds(s * SL, SL)])


_deg_kernel = pl.kernel(
    _deg_body,
    out_type=jax.ShapeDtypeStruct((NC, 1, NPAD), jnp.float32),
    mesh=_mesh,
    compiler_params=_sc_params,
    scratch_types=[
        pltpu.VMEM_SHARED((NPAD,), jnp.float32),
        pltpu.VMEM((SL,), jnp.float32),
        pltpu.VMEM((R // (NC * NT), EC), jnp.int32),
        pltpu.VMEM((EC,), jnp.float32),
    ],
)


NB = 4


def _scatter_body(npass, y2_hbm, idx4_hbm, dst2d_hbm, out_hbm,
                  tab, acc, sidx, didx, rows, gsem, ssem):
  c = lax.axis_index("c")
  s = lax.axis_index("s")
  nrows = R // NT
  base = s * nrows

  pltpu.sync_copy(dst2d_hbm.at[pl.ds(base, nrows), :], didx)

  for p in range(npass):
    v = c * npass + p
    pltpu.sync_copy(y2_hbm.at[pl.ds(v * NPAD + s * SL, SL), :],
                    tab.at[pl.ds(s * SL, SL), :])
    pltpu.sync_copy(y2_hbm.at[pl.ds(v * NPAD + s * SL, SL), :],
                    acc.at[pl.ds(s * SL, SL), :])
    pltpu.sync_copy(idx4_hbm.at[v, pl.ds(base, nrows), :], sidx)
    plsc.subcore_barrier()

    def gstart(j, q):
      pltpu.async_copy(tab.at[sidx.at[j]], rows.at[q], gsem)

    def gwait(j, q):
      pltpu.make_async_copy(tab.at[sidx.at[j]], rows.at[q], gsem).wait()

    def sstart(j, q):
      pltpu.async_copy(rows.at[q], acc.at[didx.at[j]], ssem, add=True)

    def swait(j, q):
      pltpu.make_async_copy(rows.at[q], acc.at[didx.at[j]], ssem).wait()

    for q in range(NB):
      gstart(q, q)

    def body(i, _):
      jA = 2 * NB * i
      jB = jA + NB
      for q in range(NB):
        gwait(jA + q, q)
        sstart(jA + q, q)
      for q in range(NB):
        @pl.when(i > 0)
        def _():
          swait(jA - NB + q, NB + q)

        gstart(jB + q, NB + q)
      for q in range(NB):
        gwait(jB + q, NB + q)
        sstart(jB + q, NB + q)
      for q in range(NB):
        swait(jA + q, q)

        @pl.when(jB + NB + q < nrows)
        def _():
          gstart(jB + NB + q, q)
      return 0
    lax.fori_loop(0, nrows // (2 * NB), body, 0)
    for q in range(NB):
      swait(nrows - NB + q, NB + q)

    plsc.subcore_barrier()
    pltpu.sync_copy(acc.at[pl.ds(s * SL, SL), :],
                    out_hbm.at[v, pl.ds(s * SL, SL), :])
    if p + 1 < npass:
      plsc.subcore_barrier()


@functools.cache
def _scatter_kernel(w, npass):
  return pl.kernel(
      functools.partial(_scatter_body, npass),
      out_type=jax.ShapeDtypeStruct((NC * npass, NPAD, w), jnp.float32),
      mesh=_mesh,
      compiler_params=_sc_params,
      scratch_types=[
          pltpu.VMEM_SHARED((NPAD, w), jnp.float32),
          pltpu.VMEM_SHARED((NPAD, w), jnp.float32),
          pltpu.VMEM((R // NT, EC), jnp.int32),
          pltpu.VMEM((R // NT, EC), jnp.int32),
          pltpu.VMEM((2 * NB, EC, w), jnp.float32),
          pltpu.SemaphoreType.DMA,
          pltpu.SemaphoreType.DMA,
      ],
  )


BR = 1000


def _split_panels(o_ref, y):
  np_, _, w = o_ref.shape
  for p in range(np_):
    o_ref[p] = y[:, p * w:(p + 1) * w]


def _mm1_body(x_ref, d0_ref, d1_ref, w_ref, o_ref):
  dinv = lax.rsqrt(d0_ref[...] + d1_ref[...] + 1.0)
  y = jnp.dot(x_ref[...], w_ref[...],
              preferred_element_type=jnp.float32) * dinv
  _split_panels(o_ref, y)


def _mid_body(aL_ref, aR_ref, d0_ref, d1_ref, bL_ref, bR_ref, wt_ref, wb_ref,
              o_ref):
  dinv = lax.rsqrt(d0_ref[...] + d1_ref[...] + 1.0)
  hL = jnp.maximum(aL_ref[0] * dinv + bL_ref[...], 0.0)
  hR = jnp.maximum(aR_ref[0] * dinv + bR_ref[...], 0.0)
  y = (jnp.dot(hL, wt_ref[...], preferred_element_type=jnp.float32)
       + jnp.dot(hR, wb_ref[...], preferred_element_type=jnp.float32)) * dinv
  _split_panels(o_ref, y)


def _fin_body(a0_ref, a1_ref, a2_ref, a3_ref, d0_ref, d1_ref, b_ref, o_ref):
  dinv = lax.rsqrt(d0_ref[...] + d1_ref[...] + 1.0)
  w = a0_ref.shape[2]
  pieces = [a_ref[0] * dinv + b_ref[..., p * w:(p + 1) * w]
            for p, a_ref in enumerate((a0_ref, a1_ref, a2_ref, a3_ref))]
  o_ref[...] = jnp.concatenate(pieces, axis=1)


def _row_spec(width):
  return pl.BlockSpec((BR, width), lambda i: (i, 0))


def _half_spec(half, width):
  return pl.BlockSpec((1, BR, width), lambda i, _h=half: (_h, i, 0))


def _out3_spec(npanels, width):
  return pl.BlockSpec((npanels, BR, width), lambda i: (0, i, 0))


def _whole_spec(shape):
  return pl.BlockSpec(shape, lambda i: tuple(0 for _ in shape))


def _panels_of(fout):
  return 2 if fout <= 64 else 4


def _mm1(x, d0, d1, w):
  fout = w.shape[1]
  np_ = _panels_of(fout)
  return pl.pallas_call(
      _mm1_body,
      grid=(N // BR,),
      in_specs=[_row_spec(x.shape[1]), _row_spec(1), _row_spec(1),
                _whole_spec(w.shape)],
      out_specs=_out3_spec(np_, fout // np_),
      out_shape=jax.ShapeDtypeStruct((np_, NPAD, fout // np_), jnp.float32),
  )(x, d0, d1, w)


def _mid(a, d0, d1, bL, bR, wt, wb):
  dh = a.shape[2]
  fout = wt.shape[1]
  np_ = _panels_of(fout)
  return pl.pallas_call(
      _mid_body,
      grid=(N // BR,),
      in_specs=[_half_spec(0, dh), _half_spec(1, dh),
                _row_spec(1), _row_spec(1),
                _whole_spec(bL.shape), _whole_spec(bR.shape),
                _whole_spec(wt.shape), _whole_spec(wb.shape)],
      out_specs=_out3_spec(np_, fout // np_),
      out_shape=jax.ShapeDtypeStruct((np_, NPAD, fout // np_), jnp.float32),
  )(a, a, d0, d1, bL, bR, wt, wb)


def _fin(a, d0, d1, b):
  w = a.shape[2]
  return pl.pallas_call(
      _fin_body,
      grid=(N // BR,),
      in_specs=[_half_spec(0, w), _half_spec(1, w),
                _half_spec(2, w), _half_spec(3, w),
                _row_spec(1), _row_spec(1), _whole_spec(b.shape)],
      out_specs=_row_spec(4 * w),
      out_shape=jax.ShapeDtypeStruct((N, 4 * w), jnp.float32),
  )(a, a, a, a, d0, d1, b)


@jax.jit
def _run(x, edge_index, W1, b1, W2, b2, W3, b3, W4, b4):
  src = edge_index[0]
  dst = edge_index[1]
  src_p = jnp.concatenate([src, jnp.zeros((EPAD - E,), jnp.int32)])
  dst_p = jnp.concatenate([dst, jnp.full((EPAD - E,), N, jnp.int32)])
  src2d = src_p.reshape(R, EC)
  dst2d = dst_p.reshape(R, EC)
  idx4 = jnp.stack([src2d + v * NPAD for v in range(4)])

  degp = _deg_kernel(dst2d)
  d0 = degp[0, 0, :N, None]
  d1 = degp[1, 0, :N, None]

  def scatter(y3):
    np_, _, w = y3.shape
    return _scatter_kernel(w, np_ // NC)(
        y3.reshape(np_ * NPAD, w), idx4, dst2d)

  a = scatter(_mm1(x, d0, d1, W1))
  a = scatter(_mid(a, d0, d1, b1[None, :32], b1[None, 32:], W2[:32], W2[32:]))
  a = scatter(_mid(a, d0, d1, b2[None, :16], b2[None, 16:], W3[:16], W3[16:]))
  a = scatter(_mid(a, d0, d1, b3[None, :32], b3[None, 32:], W4[:32], W4[32:]))
  return _fin(a, d0, d1, b4[None, :])


def kernel(x, edge_index, W1, b1, W2, b2, W3, b3, W4, b4):
  return _run(x, edge_index, W1, b1, W2, b2, W3, b3, W4, b4)
